# baseline (speedup 1.0000x reference)
import jax
import jax.numpy as jnp
from jax import lax
from jax.experimental import pallas as pl
from jax.experimental.pallas import tpu as pltpu
from jax.experimental.pallas import tpu_sc as plsc


def _select_last_body(flat_hbm, len_hbm, out_hbm, len_v, rows_v, sem):
    wid = lax.axis_index("s") * 2 + lax.axis_index("c")

    @pl.when(wid == 0)
    def _():
        pltpu.sync_copy(len_hbm, len_v.at[pl.ds(0, 4)])
        lens = len_v[...]
        lane = lax.iota(jnp.int32, 16)
        row = jnp.where(lens > 0, lens - 1, 4095)
        flat = jnp.where(lane < 4, lane * 4096 + row, 0)
        pltpu.async_copy(flat_hbm.at[flat], rows_v, sem).wait()
        pltpu.sync_copy(rows_v.at[pl.ds(0, 4)], out_hbm)


def kernel(input_, lengths):
    B, T, D = input_.shape
    flat = input_.reshape(B * T, D)
    lens = lengths.astype(jnp.int32)
    mesh = plsc.VectorSubcoreMesh(core_axis_name="c", subcore_axis_name="s")
    out = pl.kernel(
        _select_last_body,
        out_type=jax.ShapeDtypeStruct((B, D), input_.dtype),
        mesh=mesh,
        scratch_types=[
            pltpu.VMEM((16,), jnp.int32),
            pltpu.VMEM((16, D), jnp.float32),
            pltpu.SemaphoreType.DMA,
        ],
    )(flat, lens)
    return out[:, None, :]


# speedup vs baseline: 1.0794x; 1.0794x over previous
import jax
import jax.numpy as jnp
from jax import lax
from jax.experimental import pallas as pl
from jax.experimental.pallas import tpu as pltpu
from jax.experimental.pallas import tpu_sc as plsc


def _select_last_body(flat_hbm, len_hbm, out_hbm, len_v, rows_v, sem):
    pltpu.sync_copy(len_hbm, len_v.at[pl.ds(0, 4)])
    lens = len_v[...]
    lane = lax.iota(jnp.int32, 16)
    row = jnp.where(lens > 0, lens - 1, 4095)
    flat = jnp.where(lane < 4, lane * 4096 + row, 0)
    pltpu.async_copy(flat_hbm.at[flat], rows_v, sem).wait()
    pltpu.sync_copy(rows_v.at[pl.ds(0, 4)], out_hbm)


def kernel(input_, lengths):
    B, T, D = input_.shape
    flat = input_.reshape(B * T, D)
    lens = lengths.astype(jnp.int32)
    mesh = plsc.VectorSubcoreMesh(
        core_axis_name="c", subcore_axis_name="s", num_cores=1, num_subcores=1
    )
    out = pl.kernel(
        _select_last_body,
        out_type=jax.ShapeDtypeStruct((B, D), input_.dtype),
        mesh=mesh,
        scratch_types=[
            pltpu.VMEM((16,), jnp.int32),
            pltpu.VMEM((16, D), jnp.float32),
            pltpu.SemaphoreType.DMA,
        ],
    )(flat, lens)
    return out[:, None, :]
